# baseline (speedup 1.0000x reference)
import functools

import jax
import jax.numpy as jnp
from jax import lax
from jax.experimental import pallas as pl
from jax.experimental.pallas import tpu as pltpu
from jax.experimental.pallas import tpu_sc as plsc

N_NODES = 10000
N_EDGES = 320000
D_IN = 128
D_OUT = 128
D_EDGE = 16

NC = 2
NS = 16
NW = NC * NS
CHUNK = 88
N_CHUNKS = 116
E_PAD = NW * N_CHUNKS * CHUNK
N_PAD = 10112
ROWS_PER_S = N_PAD // NS


def _sc_aggregate(x, src1, dst1, r, z128):
    mesh = plsc.VectorSubcoreMesh(core_axis_name="c", subcore_axis_name="s")

    @functools.partial(
        pl.kernel,
        out_type=jax.ShapeDtypeStruct((NC, N_PAD, D_OUT), jnp.float32),
        mesh=mesh,
        scratch_types=(
            [pltpu.VMEM((CHUNK,), jnp.int32) for _ in range(4)]
            + [pltpu.VMEM((CHUNK,), jnp.int32) for _ in range(4)]
            + [pltpu.VMEM((CHUNK, D_OUT), jnp.float32) for _ in range(2)]
            + [pltpu.VMEM((CHUNK, D_OUT), jnp.float32) for _ in range(2)]
            + [pltpu.VMEM_SHARED((N_PAD, D_OUT), jnp.float32)]
            + [pltpu.SemaphoreType.DMA for _ in range(6)]
        ),
    )
    def k(x_hbm, src_hbm, dst_hbm, r_hbm, z128_hbm, acc_out,
          si0, si1, si2, si3, di0, di1, di2, di3,
          rows_a, rows_b, r_a, r_b, acc_sh,
          i0, i1, i2, i3, g0, g1):
        c = lax.axis_index("c")
        s = lax.axis_index("s")
        w = c * NS + s
        src_v = (si0, si1, si2, si3)
        dst_v = (di0, di1, di2, di3)
        rows_v = (rows_a, rows_b)
        r_v = (r_a, r_b)
        isem = (i0, i1, i2, i3)
        gsem = (g0, g1)

        pltpu.sync_copy(z128_hbm, acc_sh.at[pl.ds(s * ROWS_PER_S, ROWS_PER_S)])
        plsc.subcore_barrier()

        def cbase(j):
            return (w * N_CHUNKS + j) * CHUNK

        def issue_idx(j, q):
            pltpu.async_copy(src_hbm.at[pl.ds(cbase(j), CHUNK)], src_v[q],
                             isem[q])
            pltpu.async_copy(dst_hbm.at[pl.ds(cbase(j), CHUNK)], dst_v[q],
                             isem[q])

        def wait_idx(q):
            pltpu.make_async_copy(src_hbm.at[pl.ds(0, CHUNK)], src_v[q],
                                  isem[q]).wait()
            pltpu.make_async_copy(src_hbm.at[pl.ds(0, CHUNK)], dst_v[q],
                                  isem[q]).wait()

        def issue_gather(j, db, q):
            pltpu.async_copy(x_hbm.at[src_v[q]], rows_v[db], gsem[db])
            pltpu.async_copy(r_hbm.at[pl.ds(cbase(j), CHUNK)], r_v[db],
                             gsem[db])

        def wait_gather(db):
            pltpu.make_async_copy(r_hbm.at[pl.ds(0, CHUNK)], rows_v[db],
                                  gsem[db]).wait()
            pltpu.make_async_copy(r_hbm.at[pl.ds(0, CHUNK)], r_v[db],
                                  gsem[db]).wait()

        def scatter(db, q):
            pltpu.sync_copy(rows_v[db], acc_sh.at[dst_v[q]], add=True)
            pltpu.sync_copy(r_v[db], acc_sh.at[dst_v[q]], add=True)

        def step(j, off, do_idx):
            b = off % 2
            wait_idx((off + 1) % 4)
            issue_gather(j + 1, 1 - b, (off + 1) % 4)
            wait_gather(b)
            scatter(b, off % 4)
            if do_idx:
                issue_idx(j + 3, (off + 3) % 4)

        issue_idx(0, 0)
        wait_idx(0)
        issue_gather(0, 0, 0)
        issue_idx(1, 1)
        issue_idx(2, 2)

        def quad(k_, carry):
            j0 = 4 * k_
            for off in range(4):
                step(j0 + off, off, True)
            return carry

        lax.fori_loop(0, N_CHUNKS // 4 - 1, quad, 0)

        step(N_CHUNKS - 4, 0, True)
        step(N_CHUNKS - 3, 1, False)
        step(N_CHUNKS - 2, 2, False)
        wait_gather(1)
        scatter(1, 3)
        plsc.subcore_barrier()

        pltpu.sync_copy(acc_sh.at[pl.ds(s * ROWS_PER_S, ROWS_PER_S)],
                        acc_out.at[c, pl.ds(s * ROWS_PER_S, ROWS_PER_S)])

    return k(x, src1, dst1, r, z128)


def _pre_x_body(feats_ref, wr_ref, br_ref, x_ref):
    x_ref[...] = jnp.dot(feats_ref[...], wr_ref[...],
                         preferred_element_type=jnp.float32) + br_ref[...]


def _pre_r_body(ea_ref, we_ref, be_ref, r_ref):
    r_ref[...] = jnp.dot(ea_ref[...], we_ref[...],
                         preferred_element_type=jnp.float32) + be_ref[...]


def _post_body(acc_ref, feats_ref, wres_ref, bres_ref, gamma_ref, beta_ref,
               out_ref):
    agg = (acc_ref[0] + acc_ref[1])[:N_NODES]
    res = jnp.dot(feats_ref[...], wres_ref[...],
                  preferred_element_type=jnp.float32) + bres_ref[...]
    y = jnp.maximum(agg, 0.0) + jnp.maximum(res, 0.0)
    mean = jnp.mean(y, axis=0, keepdims=True)
    var = jnp.mean((y - mean) ** 2, axis=0, keepdims=True)
    out_ref[...] = ((y - mean) * lax.rsqrt(var + 1e-5) * gamma_ref[...]
                    + beta_ref[...])


R_BLK = E_PAD // 32


def kernel(feats, edge_index, edge_attr, W_rel, b_rel, W_edge, b_edge,
           W_res, b_res, gamma, beta):
    ei = edge_index.astype(jnp.int32)
    npad = E_PAD - N_EDGES
    src1 = jnp.concatenate([ei[0], jnp.zeros((npad,), jnp.int32)])
    dst1 = jnp.concatenate([ei[1], jnp.full((npad,), N_PAD - 1, jnp.int32)])
    ea_p = jnp.pad(edge_attr, ((0, npad), (0, 0)))
    z128 = jnp.zeros((ROWS_PER_S, D_OUT), jnp.float32)

    x = pl.pallas_call(
        _pre_x_body,
        out_shape=jax.ShapeDtypeStruct((N_NODES, D_OUT), jnp.float32),
    )(feats, W_rel, b_rel.reshape(1, D_OUT))

    r = pl.pallas_call(
        _pre_r_body,
        grid=(E_PAD // R_BLK,),
        in_specs=[
            pl.BlockSpec((R_BLK, D_EDGE), lambda i: (i, 0)),
            pl.BlockSpec((D_EDGE, D_OUT), lambda i: (0, 0)),
            pl.BlockSpec((1, D_OUT), lambda i: (0, 0)),
        ],
        out_specs=pl.BlockSpec((R_BLK, D_OUT), lambda i: (i, 0)),
        out_shape=jax.ShapeDtypeStruct((E_PAD, D_OUT), jnp.float32),
    )(ea_p, W_edge, b_edge.reshape(1, D_OUT))

    acc = _sc_aggregate(x, src1, dst1, r, z128)

    out = pl.pallas_call(
        _post_body,
        out_shape=jax.ShapeDtypeStruct((N_NODES, D_OUT), jnp.float32),
    )(acc, feats, W_res, b_res.reshape(1, D_OUT),
      gamma.reshape(1, D_OUT), beta.reshape(1, D_OUT))
    return out


# speedup vs baseline: 1.0089x; 1.0089x over previous
import functools

import jax
import jax.numpy as jnp
from jax import lax
from jax.experimental import pallas as pl
from jax.experimental.pallas import tpu as pltpu
from jax.experimental.pallas import tpu_sc as plsc

N_NODES = 10000
N_EDGES = 320000
D_IN = 128
D_OUT = 128
D_EDGE = 16

NC = 2
NS = 16
NW = NC * NS
CHUNK = 88
N_CHUNKS = 116
E_PAD = NW * N_CHUNKS * CHUNK
N_PAD = 10112
ROWS_PER_S = N_PAD // NS


def _sc_aggregate(x, src1, dst1, r, z128):
    mesh = plsc.VectorSubcoreMesh(core_axis_name="c", subcore_axis_name="s")

    @functools.partial(
        pl.kernel,
        out_type=jax.ShapeDtypeStruct((NC, N_PAD, D_OUT), jnp.float32),
        mesh=mesh,
        scratch_types=(
            [pltpu.VMEM((CHUNK,), jnp.int32) for _ in range(4)]
            + [pltpu.VMEM((CHUNK,), jnp.int32) for _ in range(4)]
            + [pltpu.VMEM((CHUNK, D_OUT), jnp.float32) for _ in range(2)]
            + [pltpu.VMEM((CHUNK, D_OUT), jnp.float32) for _ in range(2)]
            + [pltpu.VMEM_SHARED((N_PAD, D_OUT), jnp.float32)]
            + [pltpu.SemaphoreType.DMA for _ in range(8)]
        ),
    )
    def k(x_hbm, src_hbm, dst_hbm, r_hbm, z128_hbm, acc_out,
          si0, si1, si2, si3, di0, di1, di2, di3,
          rows_a, rows_b, r_a, r_b, acc_sh,
          i0, i1, i2, i3, g0, g1, s0, s1):
        c = lax.axis_index("c")
        s = lax.axis_index("s")
        w = c * NS + s
        src_v = (si0, si1, si2, si3)
        dst_v = (di0, di1, di2, di3)
        rows_v = (rows_a, rows_b)
        r_v = (r_a, r_b)
        isem = (i0, i1, i2, i3)
        gsem = (g0, g1)
        ssem = (s0, s1)

        pltpu.sync_copy(z128_hbm, acc_sh.at[pl.ds(s * ROWS_PER_S, ROWS_PER_S)])
        plsc.subcore_barrier()

        def cbase(j):
            return (w * N_CHUNKS + j) * CHUNK

        def issue_idx(j, q):
            pltpu.async_copy(src_hbm.at[pl.ds(cbase(j), CHUNK)], src_v[q],
                             isem[q])
            pltpu.async_copy(dst_hbm.at[pl.ds(cbase(j), CHUNK)], dst_v[q],
                             isem[q])

        def wait_idx(q):
            pltpu.make_async_copy(src_hbm.at[pl.ds(0, CHUNK)], src_v[q],
                                  isem[q]).wait()
            pltpu.make_async_copy(src_hbm.at[pl.ds(0, CHUNK)], dst_v[q],
                                  isem[q]).wait()

        def issue_gather(j, db, q):
            pltpu.async_copy(x_hbm.at[src_v[q]], rows_v[db], gsem[db])
            pltpu.async_copy(r_hbm.at[pl.ds(cbase(j), CHUNK)], r_v[db],
                             gsem[db])

        def wait_gather(db):
            pltpu.make_async_copy(r_hbm.at[pl.ds(0, CHUNK)], rows_v[db],
                                  gsem[db]).wait()
            pltpu.make_async_copy(r_hbm.at[pl.ds(0, CHUNK)], r_v[db],
                                  gsem[db]).wait()

        def issue_scatter(db, q):
            pltpu.async_copy(rows_v[db], acc_sh.at[dst_v[q]], ssem[db],
                             add=True)
            pltpu.async_copy(r_v[db], acc_sh.at[dst_v[q]], ssem[db],
                             add=True)

        def wait_scatter(db):
            pltpu.make_async_copy(r_hbm.at[pl.ds(0, CHUNK)], rows_v[db],
                                  ssem[db]).wait()
            pltpu.make_async_copy(r_hbm.at[pl.ds(0, CHUNK)], r_v[db],
                                  ssem[db]).wait()

        def step(j, off, do_idx):
            b = off % 2
            wait_idx((off + 1) % 4)
            wait_scatter(1 - b)
            issue_gather(j + 1, 1 - b, (off + 1) % 4)
            wait_gather(b)
            issue_scatter(b, off % 4)
            if do_idx:
                issue_idx(j + 3, (off + 3) % 4)

        issue_idx(0, 0)
        wait_idx(0)
        issue_gather(0, 0, 0)
        issue_idx(1, 1)
        issue_idx(2, 2)
        wait_idx(1)
        issue_gather(1, 1, 1)
        wait_gather(0)
        issue_scatter(0, 0)
        issue_idx(3, 3)

        def quad(k_, carry):
            j0 = 1 + 4 * k_
            for t in range(4):
                step(j0 + t, (1 + t) % 4, True)
            return carry

        lax.fori_loop(0, (N_CHUNKS - 4) // 4, quad, 0)

        step(N_CHUNKS - 3, (N_CHUNKS - 3) % 4, False)
        step(N_CHUNKS - 2, (N_CHUNKS - 2) % 4, False)
        wait_scatter(0)
        wait_gather(1)
        issue_scatter(1, (N_CHUNKS - 1) % 4)
        wait_scatter(1)
        plsc.subcore_barrier()

        pltpu.sync_copy(acc_sh.at[pl.ds(s * ROWS_PER_S, ROWS_PER_S)],
                        acc_out.at[c, pl.ds(s * ROWS_PER_S, ROWS_PER_S)])

    return k(x, src1, dst1, r, z128)


def _pre_x_body(feats_ref, wr_ref, br_ref, x_ref):
    x_ref[...] = jnp.dot(feats_ref[...], wr_ref[...],
                         preferred_element_type=jnp.float32) + br_ref[...]


def _pre_r_body(ea_ref, we_ref, be_ref, r_ref):
    r_ref[...] = jnp.dot(ea_ref[...], we_ref[...],
                         preferred_element_type=jnp.float32) + be_ref[...]


def _post_body(acc_ref, feats_ref, wres_ref, bres_ref, gamma_ref, beta_ref,
               out_ref):
    agg = (acc_ref[0] + acc_ref[1])[:N_NODES]
    res = jnp.dot(feats_ref[...], wres_ref[...],
                  preferred_element_type=jnp.float32) + bres_ref[...]
    y = jnp.maximum(agg, 0.0) + jnp.maximum(res, 0.0)
    mean = jnp.mean(y, axis=0, keepdims=True)
    var = jnp.mean((y - mean) ** 2, axis=0, keepdims=True)
    out_ref[...] = ((y - mean) * lax.rsqrt(var + 1e-5) * gamma_ref[...]
                    + beta_ref[...])


R_BLK = E_PAD // 32


def kernel(feats, edge_index, edge_attr, W_rel, b_rel, W_edge, b_edge,
           W_res, b_res, gamma, beta):
    ei = edge_index.astype(jnp.int32)
    npad = E_PAD - N_EDGES
    src1 = jnp.concatenate([ei[0], jnp.zeros((npad,), jnp.int32)])
    dst1 = jnp.concatenate([ei[1], jnp.full((npad,), N_PAD - 1, jnp.int32)])
    ea_p = jnp.pad(edge_attr, ((0, npad), (0, 0)))
    z128 = jnp.zeros((ROWS_PER_S, D_OUT), jnp.float32)

    x = pl.pallas_call(
        _pre_x_body,
        out_shape=jax.ShapeDtypeStruct((N_NODES, D_OUT), jnp.float32),
    )(feats, W_rel, b_rel.reshape(1, D_OUT))

    r = pl.pallas_call(
        _pre_r_body,
        grid=(E_PAD // R_BLK,),
        in_specs=[
            pl.BlockSpec((R_BLK, D_EDGE), lambda i: (i, 0)),
            pl.BlockSpec((D_EDGE, D_OUT), lambda i: (0, 0)),
            pl.BlockSpec((1, D_OUT), lambda i: (0, 0)),
        ],
        out_specs=pl.BlockSpec((R_BLK, D_OUT), lambda i: (i, 0)),
        out_shape=jax.ShapeDtypeStruct((E_PAD, D_OUT), jnp.float32),
    )(ea_p, W_edge, b_edge.reshape(1, D_OUT))

    acc = _sc_aggregate(x, src1, dst1, r, z128)

    out = pl.pallas_call(
        _post_body,
        out_shape=jax.ShapeDtypeStruct((N_NODES, D_OUT), jnp.float32),
    )(acc, feats, W_res, b_res.reshape(1, D_OUT),
      gamma.reshape(1, D_OUT), beta.reshape(1, D_OUT))
    return out


# speedup vs baseline: 1.0268x; 1.0178x over previous
import functools

import jax
import jax.numpy as jnp
from jax import lax
from jax.experimental import pallas as pl
from jax.experimental.pallas import tpu as pltpu
from jax.experimental.pallas import tpu_sc as plsc

N_NODES = 10000
N_EDGES = 320000
D_IN = 128
D_OUT = 128
D_EDGE = 16

NC = 2
NS = 16
NW = NC * NS
CHUNK = 88
N_CHUNKS = 116
E_PAD = NW * N_CHUNKS * CHUNK
N_PAD = 10112
ROWS_PER_S = N_PAD // NS


def _sc_aggregate(x, src1, dst1, r, z128):
    mesh = plsc.VectorSubcoreMesh(core_axis_name="c", subcore_axis_name="s")

    @functools.partial(
        pl.kernel,
        out_type=jax.ShapeDtypeStruct((NC, N_PAD, D_OUT), jnp.float32),
        mesh=mesh,
        scratch_types=(
            [pltpu.VMEM((CHUNK,), jnp.int32) for _ in range(4)]
            + [pltpu.VMEM((CHUNK,), jnp.int32) for _ in range(4)]
            + [pltpu.VMEM((CHUNK, D_OUT), jnp.float32) for _ in range(2)]
            + [pltpu.VMEM((CHUNK, D_OUT), jnp.float32) for _ in range(2)]
            + [pltpu.VMEM_SHARED((N_PAD, D_OUT), jnp.float32)]
            + [pltpu.SemaphoreType.DMA for _ in range(8)]
        ),
    )
    def k(x_hbm, src_hbm, dst_hbm, r_hbm, z128_hbm, acc_out,
          si0, si1, si2, si3, di0, di1, di2, di3,
          rows_a, rows_b, r_a, r_b, acc_sh,
          i0, i1, i2, i3, g0, g1, s0, s1):
        c = lax.axis_index("c")
        s = lax.axis_index("s")
        w = c * NS + s
        src_v = (si0, si1, si2, si3)
        dst_v = (di0, di1, di2, di3)
        rows_v = (rows_a, rows_b)
        r_v = (r_a, r_b)
        isem = (i0, i1, i2, i3)
        gsem = (g0, g1)
        ssem = (s0, s1)

        pltpu.sync_copy(z128_hbm, acc_sh.at[pl.ds(s * ROWS_PER_S, ROWS_PER_S)])
        plsc.subcore_barrier()

        def cbase(j):
            return (w * N_CHUNKS + j) * CHUNK

        def issue_idx(j, q):
            pltpu.async_copy(src_hbm.at[pl.ds(cbase(j), CHUNK)], src_v[q],
                             isem[q])
            pltpu.async_copy(dst_hbm.at[pl.ds(cbase(j), CHUNK)], dst_v[q],
                             isem[q])

        def wait_idx(q):
            pltpu.make_async_copy(src_hbm.at[pl.ds(0, CHUNK)], src_v[q],
                                  isem[q]).wait()
            pltpu.make_async_copy(src_hbm.at[pl.ds(0, CHUNK)], dst_v[q],
                                  isem[q]).wait()

        def issue_gather(j, db, q):
            pltpu.async_copy(x_hbm.at[src_v[q]], rows_v[db], gsem[db])
            pltpu.async_copy(r_hbm.at[pl.ds(cbase(j), CHUNK)], r_v[db],
                             gsem[db])

        def wait_gather(db):
            pltpu.make_async_copy(r_hbm.at[pl.ds(0, CHUNK)], rows_v[db],
                                  gsem[db]).wait()
            pltpu.make_async_copy(r_hbm.at[pl.ds(0, CHUNK)], r_v[db],
                                  gsem[db]).wait()

        def issue_scatter(db, q):
            pltpu.async_copy(rows_v[db], acc_sh.at[dst_v[q]], ssem[db],
                             add=True)
            pltpu.async_copy(r_v[db], acc_sh.at[dst_v[q]], ssem[db],
                             add=True)

        def wait_scatter(db):
            pltpu.make_async_copy(r_hbm.at[pl.ds(0, CHUNK)], rows_v[db],
                                  ssem[db]).wait()
            pltpu.make_async_copy(r_hbm.at[pl.ds(0, CHUNK)], r_v[db],
                                  ssem[db]).wait()

        def step(j, off, do_idx):
            b = off % 2
            wait_idx((off + 1) % 4)
            wait_scatter(1 - b)
            issue_gather(j + 1, 1 - b, (off + 1) % 4)
            wait_gather(b)
            issue_scatter(b, off % 4)
            if do_idx:
                issue_idx(j + 3, (off + 3) % 4)

        issue_idx(0, 0)
        wait_idx(0)
        issue_gather(0, 0, 0)
        issue_idx(1, 1)
        issue_idx(2, 2)
        wait_idx(1)
        issue_gather(1, 1, 1)
        wait_gather(0)
        issue_scatter(0, 0)
        issue_idx(3, 3)

        def quad(k_, carry):
            j0 = 1 + 4 * k_
            for t in range(4):
                step(j0 + t, (1 + t) % 4, True)
            return carry

        lax.fori_loop(0, (N_CHUNKS - 4) // 4, quad, 0)

        step(N_CHUNKS - 3, (N_CHUNKS - 3) % 4, False)
        step(N_CHUNKS - 2, (N_CHUNKS - 2) % 4, False)
        wait_scatter(0)
        wait_gather(1)
        issue_scatter(1, (N_CHUNKS - 1) % 4)
        wait_scatter(1)
        plsc.subcore_barrier()

        pltpu.sync_copy(acc_sh.at[pl.ds(s * ROWS_PER_S, ROWS_PER_S)],
                        acc_out.at[c, pl.ds(s * ROWS_PER_S, ROWS_PER_S)])

    return k(x, src1, dst1, r, z128)


def _pre_x_body(feats_ref, wr_ref, br_ref, x_ref):
    x_ref[...] = jnp.dot(feats_ref[...], wr_ref[...],
                         preferred_element_type=jnp.float32) + br_ref[...]


def _pre_r_body(ea_ref, we_ref, be_ref, r_ref):
    r_ref[...] = jnp.dot(ea_ref[...], we_ref[...],
                         preferred_element_type=jnp.float32) + be_ref[...]


def _post_body(acc_ref, feats_ref, wres_ref, bres_ref, gamma_ref, beta_ref,
               out_ref):
    agg = (acc_ref[0] + acc_ref[1])[:N_NODES]
    res = jnp.dot(feats_ref[...], wres_ref[...],
                  preferred_element_type=jnp.float32) + bres_ref[...]
    y = jnp.maximum(agg, 0.0) + jnp.maximum(res, 0.0)
    mean = jnp.mean(y, axis=0, keepdims=True)
    var = jnp.mean((y - mean) ** 2, axis=0, keepdims=True)
    out_ref[...] = ((y - mean) * lax.rsqrt(var + 1e-5) * gamma_ref[...]
                    + beta_ref[...])


R_BLK = E_PAD // 32


def kernel(feats, edge_index, edge_attr, W_rel, b_rel, W_edge, b_edge,
           W_res, b_res, gamma, beta):
    ei = edge_index.astype(jnp.int32)
    npad = E_PAD - N_EDGES
    src1 = jnp.concatenate([ei[0], jnp.zeros((npad,), jnp.int32)])
    sink = N_NODES + (jnp.arange(npad, dtype=jnp.int32) % (N_PAD - N_NODES))
    dst1 = jnp.concatenate([ei[1], sink])
    ea_p = jnp.pad(edge_attr, ((0, npad), (0, 0)))
    z128 = jnp.zeros((ROWS_PER_S, D_OUT), jnp.float32)

    x = pl.pallas_call(
        _pre_x_body,
        out_shape=jax.ShapeDtypeStruct((N_NODES, D_OUT), jnp.float32),
    )(feats, W_rel, b_rel.reshape(1, D_OUT))

    r = pl.pallas_call(
        _pre_r_body,
        grid=(E_PAD // R_BLK,),
        in_specs=[
            pl.BlockSpec((R_BLK, D_EDGE), lambda i: (i, 0)),
            pl.BlockSpec((D_EDGE, D_OUT), lambda i: (0, 0)),
            pl.BlockSpec((1, D_OUT), lambda i: (0, 0)),
        ],
        out_specs=pl.BlockSpec((R_BLK, D_OUT), lambda i: (i, 0)),
        out_shape=jax.ShapeDtypeStruct((E_PAD, D_OUT), jnp.float32),
    )(ea_p, W_edge, b_edge.reshape(1, D_OUT))

    acc = _sc_aggregate(x, src1, dst1, r, z128)

    out = pl.pallas_call(
        _post_body,
        out_shape=jax.ShapeDtypeStruct((N_NODES, D_OUT), jnp.float32),
    )(acc, feats, W_res, b_res.reshape(1, D_OUT),
      gamma.reshape(1, D_OUT), beta.reshape(1, D_OUT))
    return out
